# baseline (speedup 1.0000x reference)
import functools
import math

import jax
import jax.numpy as jnp
from jax import lax
from jax.experimental import pallas as pl
from jax.experimental.pallas import tpu as pltpu
from jax.experimental.pallas import tpu_sc as plsc

N, D, B, K, NL, NH = 20000, 768, 1024, 16, 2, 2
R = K + 2
DH = D // NH
L = 16
NC, NS = 2, 16
NW = NC * NS
BPW = B // NW
E = 4
CH = BPW // E
IDXC = E * R


_sc_mesh = plsc.VectorSubcoreMesh(core_axis_name="c", subcore_axis_name="s")


@functools.partial(
    pl.kernel,
    mesh=_sc_mesh,
    out_type=jax.ShapeDtypeStruct((B, D), jnp.float32),
    scratch_types=[
        pltpu.VMEM((2, IDXC), jnp.int32),
        pltpu.VMEM((IDXC, D), jnp.float32),
        pltpu.VMEM((IDXC, D), jnp.float32),
        pltpu.VMEM((E, D), jnp.float32),
        pltpu.SemaphoreType.DMA,
        pltpu.SemaphoreType.DMA,
    ],
)
def _sc_gather_combine(table_hbm, idx_hbm, out_hbm, idx_v, rows0, rows1,
                       acc_v, sem0, sem1):
    wid = lax.axis_index("s") * NC + lax.axis_index("c")
    rows = (rows0, rows1)
    sems = (sem0, sem1)

    def issue(ci, b):
        pltpu.sync_copy(idx_hbm.at[wid * CH + ci], idx_v.at[b])
        pltpu.async_copy(table_hbm.at[idx_v.at[b]], rows[b], sems[b])

    issue(0, 0)

    def pair(g, _):
        for b in range(2):
            ci = 2 * g + b

            @pl.when(ci + 1 < CH)
            def _():
                issue(ci + 1, 1 - b)

            pltpu.make_async_copy(
                table_hbm.at[idx_v.at[b]], rows[b], sems[b]).wait()

            def lanes(c, _):
                off = c * L
                for e in range(E):
                    b0 = e * R
                    s = (rows[b][b0, pl.ds(off, L)]
                         + rows[b][b0 + 1, pl.ds(off, L)])
                    n = rows[b][b0 + 2, pl.ds(off, L)]
                    for j in range(3, R):
                        n = n + rows[b][b0 + j, pl.ds(off, L)]
                    acc_v[e, pl.ds(off, L)] = s + n * (1.0 / K)
                return 0

            lax.fori_loop(0, D // L, lanes, 0)
            pltpu.sync_copy(acc_v, out_hbm.at[pl.ds((wid * BPW + ci * E), E)])
        return 0

    lax.fori_loop(0, CH // 2, pair, 0)


def _layernorm(x, s, b):
    m = x.mean(-1, keepdims=True)
    v = x.var(-1, keepdims=True)
    return (x - m) / jnp.sqrt(v + 1e-5) * s + b


def _embed_body(g_ref, w_ref, b_ref, t_ref, o_ref):
    x = jnp.tanh(
        jnp.dot(g_ref[...].astype(jnp.bfloat16), w_ref[...],
                preferred_element_type=jnp.float32)
        + b_ref[...]
    )
    i = lax.broadcasted_iota(jnp.int32, (B, D), 1)
    expo = (2.0 * (i // 2).astype(jnp.float32)) / D
    inv = jnp.exp(-math.log(10000.0) * expo)
    ang = (t_ref[...] / 100.0) * inv
    enc = jnp.where(i % 2 == 0, jnp.sin(ang), jnp.cos(ang))
    o_ref[...] = x + enc


_embed = pl.pallas_call(
    _embed_body,
    out_shape=jax.ShapeDtypeStruct((B, D), jnp.float32),
)


def _layer_body(h_ref, wqkv_ref, bqkv_ref, wo_ref, bo_ref, wf1_ref, bf1_ref,
                wf2_ref, bf2_ref, l1s_ref, l1b_ref, l2s_ref, l2b_ref, o_ref):
    h = h_ref[...]
    x = _layernorm(h, l1s_ref[...], l1b_ref[...]).astype(jnp.bfloat16)
    qkv = (jnp.dot(x, wqkv_ref[...], preferred_element_type=jnp.float32)
           + bqkv_ref[...])
    ri = lax.broadcasted_iota(jnp.int32, (B, B), 0)
    ci = lax.broadcasted_iota(jnp.int32, (B, B), 1)
    causal = ri >= ci
    parts = []
    for hh in range(NH):
        q = qkv[:, hh * DH:(hh + 1) * DH].astype(jnp.bfloat16)
        k = qkv[:, D + hh * DH:D + (hh + 1) * DH].astype(jnp.bfloat16)
        v = qkv[:, 2 * D + hh * DH:2 * D + (hh + 1) * DH].astype(jnp.bfloat16)
        sc = lax.dot_general(q, k, (((1,), (1,)), ((), ())),
                             preferred_element_type=jnp.float32)
        sc = sc * (1.0 / math.sqrt(float(DH)))
        sc = jnp.where(causal, sc, -1e9)
        sc = sc - sc.max(-1, keepdims=True)
        p = jnp.exp(sc)
        a = (p / p.sum(-1, keepdims=True)).astype(jnp.bfloat16)
        parts.append(jnp.dot(a, v, preferred_element_type=jnp.float32))
    o = jnp.concatenate(parts, axis=1).astype(jnp.bfloat16)
    h = h + jnp.dot(o, wo_ref[...], preferred_element_type=jnp.float32) + bo_ref[...]
    y = _layernorm(h, l2s_ref[...], l2b_ref[...]).astype(jnp.bfloat16)
    f = jnp.maximum(
        jnp.dot(y, wf1_ref[...], preferred_element_type=jnp.float32) + bf1_ref[...],
        0.0).astype(jnp.bfloat16)
    h = h + jnp.dot(f, wf2_ref[...], preferred_element_type=jnp.float32) + bf2_ref[...]
    o_ref[...] = h


_layer = pl.pallas_call(
    _layer_body,
    out_shape=jax.ShapeDtypeStruct((B, D), jnp.float32),
)


def _softplus(x):
    return jnp.maximum(x, 0.0) + jnp.log1p(jnp.exp(-jnp.abs(x)))


def _head_body(h_ref, t_ref, wh_ref, bh_ref, alpha_ref, beta_ref,
               fw_ref, fb_ref, cls_ref, ell_ref, nll_ref):
    h = h_ref[...]
    t = t_ref[...]
    lam_in = (jnp.dot(h, wh_ref[...], preferred_element_type=jnp.float32)
              + bh_ref[...])
    tprev = jnp.concatenate([t[0:1], t[0:B - 1]], axis=0)
    dt = t - tprev
    idx = lax.broadcasted_iota(jnp.int32, (B, 1), 0)
    valid = idx >= 2
    lam = _softplus(lam_in + alpha_ref[...] * dt)
    event_ll = jnp.sum(jnp.where(valid, jnp.log(lam + 1e-9), 0.0))
    sp_beta = _softplus(beta_ref[...])
    non_event_ll = jnp.sum(jnp.where(valid, sp_beta * lam * dt, 0.0))
    hl = jnp.tanh(h[B - 1:B, :])
    logits = jnp.dot(hl, fw_ref[...], preferred_element_type=jnp.float32) + fb_ref[...]
    logits = logits - logits.max(-1, keepdims=True)
    p = jnp.exp(logits)
    cls_ref[...] = p / p.sum(-1, keepdims=True)
    ell_ref[...] = jnp.reshape(event_ll, (1, 1))
    nll_ref[...] = jnp.reshape(non_event_ll, (1, 1))


_head = pl.pallas_call(
    _head_body,
    out_shape=[
        jax.ShapeDtypeStruct((1, 2), jnp.float32),
        jax.ShapeDtypeStruct((1, 1), jnp.float32),
        jax.ShapeDtypeStruct((1, 1), jnp.float32),
    ],
)


def kernel(table, W_emb, b_emb, Wqkv, bqkv, Wo, bo, Wf1, bf1, Wf2, bf2,
           ln1_s, ln1_b, ln2_s, ln2_b, w_h, b_h, alpha, beta, fc1_W, fc1_b,
           sources, destinations, neighbors, timestamps):
    idx2d = jnp.concatenate(
        [sources[:, None], destinations[:, None], neighbors], axis=1
    ).astype(jnp.int32).reshape(B * R // IDXC, IDXC)
    g = _sc_gather_combine(table, idx2d)

    t_f = timestamps.astype(jnp.float32).reshape(B, 1)
    bf = jnp.bfloat16
    h = _embed(g, W_emb.astype(bf), b_emb.reshape(1, D), t_f)
    for l in range(NL):
        h = _layer(h, Wqkv[l].astype(bf), bqkv[l].reshape(1, 3 * D),
                   Wo[l].astype(bf),
                   bo[l].reshape(1, D), Wf1[l].astype(bf),
                   bf1[l].reshape(1, D),
                   Wf2[l].astype(bf), bf2[l].reshape(1, D),
                   ln1_s[l].reshape(1, D),
                   ln1_b[l].reshape(1, D), ln2_s[l].reshape(1, D),
                   ln2_b[l].reshape(1, D))
    cls, ell, nll = _head(h, t_f, w_h, b_h.reshape(1, 1),
                          alpha.reshape(1, 1), beta.reshape(1, 1),
                          fc1_W, fc1_b.reshape(1, 2))
    return cls, ell.reshape(()), nll.reshape(())


# speedup vs baseline: 1.0152x; 1.0152x over previous
import functools
import math

import jax
import jax.numpy as jnp
from jax import lax
from jax.experimental import pallas as pl
from jax.experimental.pallas import tpu as pltpu
from jax.experimental.pallas import tpu_sc as plsc

N, D, B, K, NL, NH = 20000, 768, 1024, 16, 2, 2
R = K + 2
DH = D // NH
L = 16
NC, NS = 2, 16
NW = NC * NS
BPW = B // NW
E = 4
CH = BPW // E
IDXC = E * R


_sc_mesh = plsc.VectorSubcoreMesh(core_axis_name="c", subcore_axis_name="s")


@functools.partial(
    pl.kernel,
    mesh=_sc_mesh,
    out_type=jax.ShapeDtypeStruct((B, D), jnp.float32),
    scratch_types=[
        pltpu.VMEM((CH, IDXC), jnp.int32),
        pltpu.VMEM((IDXC, D), jnp.float32),
        pltpu.VMEM((IDXC, D), jnp.float32),
        pltpu.VMEM((E, D), jnp.float32),
        pltpu.VMEM((E, D), jnp.float32),
        pltpu.SemaphoreType.DMA,
        pltpu.SemaphoreType.DMA,
        pltpu.SemaphoreType.DMA,
        pltpu.SemaphoreType.DMA,
    ],
)
def _sc_gather_combine(table_hbm, idx_hbm, out_hbm, idx_v, rows0, rows1,
                       acc0, acc1, semg0, semg1, semo0, semo1):
    wid = lax.axis_index("s") * NC + lax.axis_index("c")
    rows = (rows0, rows1)
    accs = (acc0, acc1)
    semg = (semg0, semg1)
    semo = (semo0, semo1)

    pltpu.sync_copy(idx_hbm.at[pl.ds(wid * CH, CH)], idx_v)
    pltpu.async_copy(table_hbm.at[idx_v.at[0]], rows0, semg0)

    def pair(g, _):
        for b in range(2):
            ci = 2 * g + b

            @pl.when(ci + 1 < CH)
            def _():
                pltpu.async_copy(
                    table_hbm.at[idx_v.at[ci + 1]], rows[1 - b], semg[1 - b])

            pltpu.make_async_copy(
                table_hbm.at[idx_v.at[ci]], rows[b], semg[b]).wait()

            @pl.when(ci >= 2)
            def _():
                pltpu.make_async_copy(
                    accs[b], out_hbm.at[pl.ds(wid * BPW, E)], semo[b]).wait()

            def lanes(c, _):
                off = c * L
                for e in range(E):
                    b0 = e * R
                    s = (rows[b][b0, pl.ds(off, L)]
                         + rows[b][b0 + 1, pl.ds(off, L)])
                    n = rows[b][b0 + 2, pl.ds(off, L)]
                    for j in range(3, R):
                        n = n + rows[b][b0 + j, pl.ds(off, L)]
                    accs[b][e, pl.ds(off, L)] = s + n * (1.0 / K)
                return 0

            lax.fori_loop(0, D // L, lanes, 0)
            pltpu.async_copy(
                accs[b], out_hbm.at[pl.ds((wid * BPW + ci * E), E)], semo[b])
        return 0

    lax.fori_loop(0, CH // 2, pair, 0)
    for b in range(2):
        pltpu.make_async_copy(
            accs[b], out_hbm.at[pl.ds(wid * BPW, E)], semo[b]).wait()


def _layernorm(x, s, b):
    m = x.mean(-1, keepdims=True)
    v = x.var(-1, keepdims=True)
    return (x - m) / jnp.sqrt(v + 1e-5) * s + b


def _embed_body(g_ref, w_ref, b_ref, t_ref, o_ref):
    x = jnp.tanh(
        jnp.dot(g_ref[...].astype(jnp.bfloat16), w_ref[...],
                preferred_element_type=jnp.float32)
        + b_ref[...]
    )
    i = lax.broadcasted_iota(jnp.int32, (B, D), 1)
    expo = (2.0 * (i // 2).astype(jnp.float32)) / D
    inv = jnp.exp(-math.log(10000.0) * expo)
    ang = (t_ref[...] / 100.0) * inv
    enc = jnp.where(i % 2 == 0, jnp.sin(ang), jnp.cos(ang))
    o_ref[...] = x + enc


_embed = pl.pallas_call(
    _embed_body,
    out_shape=jax.ShapeDtypeStruct((B, D), jnp.float32),
)


def _layer_body(h_ref, wqkv_ref, bqkv_ref, wo_ref, bo_ref, wf1_ref, bf1_ref,
                wf2_ref, bf2_ref, l1s_ref, l1b_ref, l2s_ref, l2b_ref, o_ref):
    h = h_ref[...]
    x = _layernorm(h, l1s_ref[...], l1b_ref[...]).astype(jnp.bfloat16)
    qkv = (jnp.dot(x, wqkv_ref[...], preferred_element_type=jnp.float32)
           + bqkv_ref[...])
    ri = lax.broadcasted_iota(jnp.int32, (B, B), 0)
    ci = lax.broadcasted_iota(jnp.int32, (B, B), 1)
    causal = ri >= ci
    parts = []
    for hh in range(NH):
        q = qkv[:, hh * DH:(hh + 1) * DH].astype(jnp.bfloat16)
        k = qkv[:, D + hh * DH:D + (hh + 1) * DH].astype(jnp.bfloat16)
        v = qkv[:, 2 * D + hh * DH:2 * D + (hh + 1) * DH].astype(jnp.bfloat16)
        sc = lax.dot_general(q, k, (((1,), (1,)), ((), ())),
                             preferred_element_type=jnp.float32)
        sc = sc * (1.0 / math.sqrt(float(DH)))
        sc = jnp.where(causal, sc, -1e9)
        sc = sc - sc.max(-1, keepdims=True)
        p = jnp.exp(sc)
        a = (p / p.sum(-1, keepdims=True)).astype(jnp.bfloat16)
        parts.append(jnp.dot(a, v, preferred_element_type=jnp.float32))
    o = jnp.concatenate(parts, axis=1).astype(jnp.bfloat16)
    h = h + jnp.dot(o, wo_ref[...], preferred_element_type=jnp.float32) + bo_ref[...]
    y = _layernorm(h, l2s_ref[...], l2b_ref[...]).astype(jnp.bfloat16)
    f = jnp.maximum(
        jnp.dot(y, wf1_ref[...], preferred_element_type=jnp.float32) + bf1_ref[...],
        0.0).astype(jnp.bfloat16)
    h = h + jnp.dot(f, wf2_ref[...], preferred_element_type=jnp.float32) + bf2_ref[...]
    o_ref[...] = h


_layer = pl.pallas_call(
    _layer_body,
    out_shape=jax.ShapeDtypeStruct((B, D), jnp.float32),
)


def _softplus(x):
    return jnp.maximum(x, 0.0) + jnp.log1p(jnp.exp(-jnp.abs(x)))


def _head_body(h_ref, t_ref, wh_ref, bh_ref, alpha_ref, beta_ref,
               fw_ref, fb_ref, cls_ref, ell_ref, nll_ref):
    h = h_ref[...]
    t = t_ref[...]
    lam_in = (jnp.dot(h, wh_ref[...], preferred_element_type=jnp.float32)
              + bh_ref[...])
    tprev = jnp.concatenate([t[0:1], t[0:B - 1]], axis=0)
    dt = t - tprev
    idx = lax.broadcasted_iota(jnp.int32, (B, 1), 0)
    valid = idx >= 2
    lam = _softplus(lam_in + alpha_ref[...] * dt)
    event_ll = jnp.sum(jnp.where(valid, jnp.log(lam + 1e-9), 0.0))
    sp_beta = _softplus(beta_ref[...])
    non_event_ll = jnp.sum(jnp.where(valid, sp_beta * lam * dt, 0.0))
    hl = jnp.tanh(h[B - 1:B, :])
    logits = jnp.dot(hl, fw_ref[...], preferred_element_type=jnp.float32) + fb_ref[...]
    logits = logits - logits.max(-1, keepdims=True)
    p = jnp.exp(logits)
    cls_ref[...] = p / p.sum(-1, keepdims=True)
    ell_ref[...] = jnp.reshape(event_ll, (1, 1))
    nll_ref[...] = jnp.reshape(non_event_ll, (1, 1))


_head = pl.pallas_call(
    _head_body,
    out_shape=[
        jax.ShapeDtypeStruct((1, 2), jnp.float32),
        jax.ShapeDtypeStruct((1, 1), jnp.float32),
        jax.ShapeDtypeStruct((1, 1), jnp.float32),
    ],
)


def kernel(table, W_emb, b_emb, Wqkv, bqkv, Wo, bo, Wf1, bf1, Wf2, bf2,
           ln1_s, ln1_b, ln2_s, ln2_b, w_h, b_h, alpha, beta, fc1_W, fc1_b,
           sources, destinations, neighbors, timestamps):
    idx2d = jnp.concatenate(
        [sources[:, None], destinations[:, None], neighbors], axis=1
    ).astype(jnp.int32).reshape(B * R // IDXC, IDXC)
    g = _sc_gather_combine(table, idx2d)

    t_f = timestamps.astype(jnp.float32).reshape(B, 1)
    bf = jnp.bfloat16
    h = _embed(g, W_emb.astype(bf), b_emb.reshape(1, D), t_f)
    for l in range(NL):
        h = _layer(h, Wqkv[l].astype(bf), bqkv[l].reshape(1, 3 * D),
                   Wo[l].astype(bf),
                   bo[l].reshape(1, D), Wf1[l].astype(bf),
                   bf1[l].reshape(1, D),
                   Wf2[l].astype(bf), bf2[l].reshape(1, D),
                   ln1_s[l].reshape(1, D),
                   ln1_b[l].reshape(1, D), ln2_s[l].reshape(1, D),
                   ln2_b[l].reshape(1, D))
    cls, ell, nll = _head(h, t_f, w_h, b_h.reshape(1, 1),
                          alpha.reshape(1, 1), beta.reshape(1, 1),
                          fc1_W, fc1_b.reshape(1, 2))
    return cls, ell.reshape(()), nll.reshape(())


# speedup vs baseline: 1.0325x; 1.0171x over previous
import functools
import math

import jax
import jax.numpy as jnp
from jax import lax
from jax.experimental import pallas as pl
from jax.experimental.pallas import tpu as pltpu
from jax.experimental.pallas import tpu_sc as plsc

N, D, B, K, NL, NH = 20000, 768, 1024, 16, 2, 2
R = K + 2
DH = D // NH
L = 16
NC, NS = 2, 16
NW = NC * NS
BPW = B // NW
E = 4
CH = BPW // E
IDXC = E * R


_sc_mesh = plsc.VectorSubcoreMesh(core_axis_name="c", subcore_axis_name="s")


@functools.partial(
    pl.kernel,
    mesh=_sc_mesh,
    out_type=jax.ShapeDtypeStruct((B, D), jnp.float32),
    scratch_types=[
        pltpu.VMEM((CH, IDXC), jnp.int32),
        pltpu.VMEM((IDXC, D), jnp.float32),
        pltpu.VMEM((IDXC, D), jnp.float32),
        pltpu.VMEM((E, D), jnp.float32),
        pltpu.VMEM((E, D), jnp.float32),
        pltpu.SemaphoreType.DMA,
        pltpu.SemaphoreType.DMA,
        pltpu.SemaphoreType.DMA,
        pltpu.SemaphoreType.DMA,
    ],
)
def _sc_gather_combine(table_hbm, idx_hbm, out_hbm, idx_v, rows0, rows1,
                       acc0, acc1, semg0, semg1, semo0, semo1):
    wid = lax.axis_index("s") * NC + lax.axis_index("c")
    rows = (rows0, rows1)
    accs = (acc0, acc1)
    semg = (semg0, semg1)
    semo = (semo0, semo1)

    pltpu.sync_copy(idx_hbm.at[pl.ds(wid * CH, CH)], idx_v)
    pltpu.async_copy(table_hbm.at[idx_v.at[0]], rows0, semg0)

    def pair(g, _):
        for b in range(2):
            ci = 2 * g + b

            @pl.when(ci + 1 < CH)
            def _():
                pltpu.async_copy(
                    table_hbm.at[idx_v.at[ci + 1]], rows[1 - b], semg[1 - b])

            pltpu.make_async_copy(
                table_hbm.at[idx_v.at[ci]], rows[b], semg[b]).wait()

            @pl.when(ci >= 2)
            def _():
                pltpu.make_async_copy(
                    accs[b], out_hbm.at[pl.ds(wid * BPW, E)], semo[b]).wait()

            def lanes(c, _):
                off = c * L
                for e in range(E):
                    b0 = e * R
                    s = (rows[b][b0, pl.ds(off, L)]
                         + rows[b][b0 + 1, pl.ds(off, L)])
                    n = rows[b][b0 + 2, pl.ds(off, L)]
                    for j in range(3, R):
                        n = n + rows[b][b0 + j, pl.ds(off, L)]
                    accs[b][e, pl.ds(off, L)] = s + n * (1.0 / K)
                return 0

            lax.fori_loop(0, D // L, lanes, 0)
            pltpu.async_copy(
                accs[b], out_hbm.at[pl.ds((wid * BPW + ci * E), E)], semo[b])
        return 0

    lax.fori_loop(0, CH // 2, pair, 0)
    for b in range(2):
        pltpu.make_async_copy(
            accs[b], out_hbm.at[pl.ds(wid * BPW, E)], semo[b]).wait()


def _layernorm(x, s, b):
    m = x.mean(-1, keepdims=True)
    v = x.var(-1, keepdims=True)
    return (x - m) / jnp.sqrt(v + 1e-5) * s + b


def _embed_body(g_ref, w_ref, b_ref, t_ref, o_ref):
    x = jnp.tanh(
        jnp.dot(g_ref[...].astype(jnp.bfloat16), w_ref[...],
                preferred_element_type=jnp.float32)
        + b_ref[...]
    )
    i = lax.broadcasted_iota(jnp.int32, (B, D), 1)
    expo = (2.0 * (i // 2).astype(jnp.float32)) / D
    inv = jnp.exp(-math.log(10000.0) * expo)
    ang = (t_ref[...] / 100.0) * inv
    enc = jnp.where(i % 2 == 0, jnp.sin(ang), jnp.cos(ang))
    o_ref[...] = x + enc


_embed = pl.pallas_call(
    _embed_body,
    out_shape=jax.ShapeDtypeStruct((B, D), jnp.float32),
)


def _layer_body(h_ref, wqkv_ref, bqkv_ref, wo_ref, bo_ref, wf1_ref, bf1_ref,
                wf2_ref, bf2_ref, l1s_ref, l1b_ref, l2s_ref, l2b_ref, o_ref):
    h = h_ref[...]
    x = _layernorm(h, l1s_ref[...], l1b_ref[...]).astype(jnp.bfloat16)
    qkv = (jnp.dot(x, wqkv_ref[...], preferred_element_type=jnp.float32)
           + bqkv_ref[...])
    ri = lax.broadcasted_iota(jnp.int32, (B, B), 0)
    ci = lax.broadcasted_iota(jnp.int32, (B, B), 1)
    causal = ri >= ci
    parts = []
    for hh in range(NH):
        q = (qkv[:, hh * DH:(hh + 1) * DH]
             * (1.0 / math.sqrt(float(DH)))).astype(jnp.bfloat16)
        k = qkv[:, D + hh * DH:D + (hh + 1) * DH].astype(jnp.bfloat16)
        v = qkv[:, 2 * D + hh * DH:2 * D + (hh + 1) * DH].astype(jnp.bfloat16)
        sc = lax.dot_general(q, k, (((1,), (1,)), ((), ())),
                             preferred_element_type=jnp.float32)
        p = jnp.where(causal, jnp.exp(sc), 0.0)
        s = p.sum(-1, keepdims=True)
        o = jnp.dot(p.astype(jnp.bfloat16), v,
                    preferred_element_type=jnp.float32)
        parts.append(o * (1.0 / s))
    o = jnp.concatenate(parts, axis=1).astype(jnp.bfloat16)
    h = h + jnp.dot(o, wo_ref[...], preferred_element_type=jnp.float32) + bo_ref[...]
    y = _layernorm(h, l2s_ref[...], l2b_ref[...]).astype(jnp.bfloat16)
    f = jnp.maximum(
        jnp.dot(y, wf1_ref[...], preferred_element_type=jnp.float32) + bf1_ref[...],
        0.0).astype(jnp.bfloat16)
    h = h + jnp.dot(f, wf2_ref[...], preferred_element_type=jnp.float32) + bf2_ref[...]
    o_ref[...] = h


_layer = pl.pallas_call(
    _layer_body,
    out_shape=jax.ShapeDtypeStruct((B, D), jnp.float32),
)


def _softplus(x):
    return jnp.maximum(x, 0.0) + jnp.log1p(jnp.exp(-jnp.abs(x)))


def _head_body(h_ref, t_ref, wh_ref, bh_ref, alpha_ref, beta_ref,
               fw_ref, fb_ref, cls_ref, ell_ref, nll_ref):
    h = h_ref[...]
    t = t_ref[...]
    lam_in = (jnp.dot(h, wh_ref[...], preferred_element_type=jnp.float32)
              + bh_ref[...])
    tprev = jnp.concatenate([t[0:1], t[0:B - 1]], axis=0)
    dt = t - tprev
    idx = lax.broadcasted_iota(jnp.int32, (B, 1), 0)
    valid = idx >= 2
    lam = _softplus(lam_in + alpha_ref[...] * dt)
    event_ll = jnp.sum(jnp.where(valid, jnp.log(lam + 1e-9), 0.0))
    sp_beta = _softplus(beta_ref[...])
    non_event_ll = jnp.sum(jnp.where(valid, sp_beta * lam * dt, 0.0))
    hl = jnp.tanh(h[B - 1:B, :])
    logits = jnp.dot(hl, fw_ref[...], preferred_element_type=jnp.float32) + fb_ref[...]
    logits = logits - logits.max(-1, keepdims=True)
    p = jnp.exp(logits)
    cls_ref[...] = p / p.sum(-1, keepdims=True)
    ell_ref[...] = jnp.reshape(event_ll, (1, 1))
    nll_ref[...] = jnp.reshape(non_event_ll, (1, 1))


_head = pl.pallas_call(
    _head_body,
    out_shape=[
        jax.ShapeDtypeStruct((1, 2), jnp.float32),
        jax.ShapeDtypeStruct((1, 1), jnp.float32),
        jax.ShapeDtypeStruct((1, 1), jnp.float32),
    ],
)


def kernel(table, W_emb, b_emb, Wqkv, bqkv, Wo, bo, Wf1, bf1, Wf2, bf2,
           ln1_s, ln1_b, ln2_s, ln2_b, w_h, b_h, alpha, beta, fc1_W, fc1_b,
           sources, destinations, neighbors, timestamps):
    idx2d = jnp.concatenate(
        [sources[:, None], destinations[:, None], neighbors], axis=1
    ).astype(jnp.int32).reshape(B * R // IDXC, IDXC)
    g = _sc_gather_combine(table, idx2d)

    t_f = timestamps.astype(jnp.float32).reshape(B, 1)
    bf = jnp.bfloat16
    h = _embed(g, W_emb.astype(bf), b_emb.reshape(1, D), t_f)
    for l in range(NL):
        h = _layer(h, Wqkv[l].astype(bf), bqkv[l].reshape(1, 3 * D),
                   Wo[l].astype(bf),
                   bo[l].reshape(1, D), Wf1[l].astype(bf),
                   bf1[l].reshape(1, D),
                   Wf2[l].astype(bf), bf2[l].reshape(1, D),
                   ln1_s[l].reshape(1, D),
                   ln1_b[l].reshape(1, D), ln2_s[l].reshape(1, D),
                   ln2_b[l].reshape(1, D))
    cls, ell, nll = _head(h, t_f, w_h, b_h.reshape(1, 1),
                          alpha.reshape(1, 1), beta.reshape(1, 1),
                          fc1_W, fc1_b.reshape(1, 2))
    return cls, ell.reshape(()), nll.reshape(())


# speedup vs baseline: 1.0834x; 1.0493x over previous
import functools
import math

import jax
import jax.numpy as jnp
from jax import lax
from jax.experimental import pallas as pl
from jax.experimental.pallas import tpu as pltpu
from jax.experimental.pallas import tpu_sc as plsc

N, D, B, K, NL, NH = 20000, 768, 1024, 16, 2, 2
R = K + 2
DH = D // NH
L = 16
NC, NS = 2, 16
NW = NC * NS
BPW = B // NW
E = 4
CH = BPW // E
IDXC = E * R


_sc_mesh = plsc.VectorSubcoreMesh(core_axis_name="c", subcore_axis_name="s")


@functools.partial(
    pl.kernel,
    mesh=_sc_mesh,
    out_type=jax.ShapeDtypeStruct((B, D), jnp.float32),
    scratch_types=[
        pltpu.VMEM((CH, IDXC), jnp.int32),
        pltpu.VMEM((IDXC, D), jnp.float32),
        pltpu.VMEM((IDXC, D), jnp.float32),
        pltpu.VMEM((E, D), jnp.float32),
        pltpu.VMEM((E, D), jnp.float32),
        pltpu.SemaphoreType.DMA,
        pltpu.SemaphoreType.DMA,
        pltpu.SemaphoreType.DMA,
        pltpu.SemaphoreType.DMA,
    ],
)
def _sc_gather_combine(table_hbm, idx_hbm, out_hbm, idx_v, rows0, rows1,
                       acc0, acc1, semg0, semg1, semo0, semo1):
    wid = lax.axis_index("s") * NC + lax.axis_index("c")
    rows = (rows0, rows1)
    accs = (acc0, acc1)
    semg = (semg0, semg1)
    semo = (semo0, semo1)

    pltpu.sync_copy(idx_hbm.at[pl.ds(wid * CH, CH)], idx_v)
    pltpu.async_copy(table_hbm.at[idx_v.at[0]], rows0, semg0)

    def pair(g, _):
        for b in range(2):
            ci = 2 * g + b

            @pl.when(ci + 1 < CH)
            def _():
                pltpu.async_copy(
                    table_hbm.at[idx_v.at[ci + 1]], rows[1 - b], semg[1 - b])

            pltpu.make_async_copy(
                table_hbm.at[idx_v.at[ci]], rows[b], semg[b]).wait()

            @pl.when(ci >= 2)
            def _():
                pltpu.make_async_copy(
                    accs[b], out_hbm.at[pl.ds(wid * BPW, E)], semo[b]).wait()

            def lanes(c, _):
                off = c * L
                for e in range(E):
                    b0 = e * R
                    s = (rows[b][b0, pl.ds(off, L)]
                         + rows[b][b0 + 1, pl.ds(off, L)])
                    n = rows[b][b0 + 2, pl.ds(off, L)]
                    for j in range(3, R):
                        n = n + rows[b][b0 + j, pl.ds(off, L)]
                    accs[b][e, pl.ds(off, L)] = s + n * (1.0 / K)
                return 0

            lax.fori_loop(0, D // L, lanes, 0)
            pltpu.async_copy(
                accs[b], out_hbm.at[pl.ds((wid * BPW + ci * E), E)], semo[b])
        return 0

    lax.fori_loop(0, CH // 2, pair, 0)
    for b in range(2):
        pltpu.make_async_copy(
            accs[b], out_hbm.at[pl.ds(wid * BPW, E)], semo[b]).wait()


def _layernorm(x, s, b):
    m = x.mean(-1, keepdims=True)
    v = x.var(-1, keepdims=True)
    return (x - m) / jnp.sqrt(v + 1e-5) * s + b


def _embed_body(g_ref, w_ref, b_ref, t_ref, o_ref):
    x = jnp.tanh(
        jnp.dot(g_ref[...].astype(jnp.bfloat16), w_ref[...],
                preferred_element_type=jnp.float32)
        + b_ref[...]
    )
    i = lax.broadcasted_iota(jnp.int32, (B, D), 1)
    expo = (2.0 * (i // 2).astype(jnp.float32)) / D
    inv = jnp.exp(-math.log(10000.0) * expo)
    ang = (t_ref[...] / 100.0) * inv
    enc = jnp.where(i % 2 == 0, jnp.sin(ang), jnp.cos(ang))
    o_ref[...] = (x + enc).astype(jnp.bfloat16)


_embed = pl.pallas_call(
    _embed_body,
    out_shape=jax.ShapeDtypeStruct((B, D), jnp.bfloat16),
)


def _encoder_layer(h, wqkv_ref, bqkv_ref, wo_ref, bo_ref, wf1_ref, bf1_ref,
                   wf2_ref, bf2_ref, l1s_ref, l1b_ref, l2s_ref, l2b_ref):
    x = _layernorm(h, l1s_ref[...], l1b_ref[...]).astype(jnp.bfloat16)
    qkv = (jnp.dot(x, wqkv_ref[...], preferred_element_type=jnp.float32)
           + bqkv_ref[...])
    ri = lax.broadcasted_iota(jnp.int32, (B, B), 0)
    ci = lax.broadcasted_iota(jnp.int32, (B, B), 1)
    causal = ri >= ci
    parts = []
    for hh in range(NH):
        q = (qkv[:, hh * DH:(hh + 1) * DH]
             * (1.0 / math.sqrt(float(DH)))).astype(jnp.bfloat16)
        k = qkv[:, D + hh * DH:D + (hh + 1) * DH].astype(jnp.bfloat16)
        v = qkv[:, 2 * D + hh * DH:2 * D + (hh + 1) * DH].astype(jnp.bfloat16)
        sc = lax.dot_general(q, k, (((1,), (1,)), ((), ())),
                             preferred_element_type=jnp.float32)
        p = jnp.where(causal, jnp.exp(sc), 0.0)
        s = p.sum(-1, keepdims=True)
        o = jnp.dot(p.astype(jnp.bfloat16), v,
                    preferred_element_type=jnp.float32)
        parts.append(o * (1.0 / s))
    o = jnp.concatenate(parts, axis=1).astype(jnp.bfloat16)
    h = h + jnp.dot(o, wo_ref[...], preferred_element_type=jnp.float32) + bo_ref[...]
    y = _layernorm(h, l2s_ref[...], l2b_ref[...]).astype(jnp.bfloat16)
    f = jnp.maximum(
        jnp.dot(y, wf1_ref[...], preferred_element_type=jnp.float32) + bf1_ref[...],
        0.0).astype(jnp.bfloat16)
    h = h + jnp.dot(f, wf2_ref[...], preferred_element_type=jnp.float32) + bf2_ref[...]
    return h


def _layer0_body(h_ref, wqkv_ref, bqkv_ref, wo_ref, bo_ref, wf1_ref, bf1_ref,
                 wf2_ref, bf2_ref, l1s_ref, l1b_ref, l2s_ref, l2b_ref, o_ref):
    h = _encoder_layer(h_ref[...].astype(jnp.float32), wqkv_ref, bqkv_ref,
                       wo_ref, bo_ref, wf1_ref, bf1_ref, wf2_ref, bf2_ref,
                       l1s_ref, l1b_ref, l2s_ref, l2b_ref)
    o_ref[...] = h.astype(jnp.bfloat16)


_layer0 = pl.pallas_call(
    _layer0_body,
    out_shape=jax.ShapeDtypeStruct((B, D), jnp.bfloat16),
)


def _softplus(x):
    return jnp.maximum(x, 0.0) + jnp.log1p(jnp.exp(-jnp.abs(x)))


def _layer1_head_body(h_ref, wqkv_ref, bqkv_ref, wo_ref, bo_ref, wf1_ref,
                      bf1_ref, wf2_ref, bf2_ref, l1s_ref, l1b_ref, l2s_ref,
                      l2b_ref, t_ref, wh_ref, bh_ref, alpha_ref, beta_ref,
                      fw_ref, fb_ref, cls_ref, ell_ref, nll_ref):
    h = _encoder_layer(h_ref[...].astype(jnp.float32), wqkv_ref, bqkv_ref,
                       wo_ref, bo_ref, wf1_ref, bf1_ref, wf2_ref, bf2_ref,
                       l1s_ref, l1b_ref, l2s_ref, l2b_ref)
    t = t_ref[...]
    lam_in = (jnp.dot(h, wh_ref[...], preferred_element_type=jnp.float32)
              + bh_ref[...])
    tprev = jnp.concatenate([t[0:1], t[0:B - 1]], axis=0)
    dt = t - tprev
    idx = lax.broadcasted_iota(jnp.int32, (B, 1), 0)
    valid = idx >= 2
    lam = _softplus(lam_in + alpha_ref[...] * dt)
    event_ll = jnp.sum(jnp.where(valid, jnp.log(lam + 1e-9), 0.0))
    sp_beta = _softplus(beta_ref[...])
    non_event_ll = jnp.sum(jnp.where(valid, sp_beta * lam * dt, 0.0))
    hl = jnp.tanh(h[B - 1:B, :])
    logits = jnp.dot(hl, fw_ref[...], preferred_element_type=jnp.float32) + fb_ref[...]
    logits = logits - logits.max(-1, keepdims=True)
    p = jnp.exp(logits)
    cls_ref[...] = p / p.sum(-1, keepdims=True)
    ell_ref[...] = jnp.reshape(event_ll, (1, 1))
    nll_ref[...] = jnp.reshape(non_event_ll, (1, 1))


_layer1_head = pl.pallas_call(
    _layer1_head_body,
    out_shape=[
        jax.ShapeDtypeStruct((1, 2), jnp.float32),
        jax.ShapeDtypeStruct((1, 1), jnp.float32),
        jax.ShapeDtypeStruct((1, 1), jnp.float32),
    ],
)


def kernel(table, W_emb, b_emb, Wqkv, bqkv, Wo, bo, Wf1, bf1, Wf2, bf2,
           ln1_s, ln1_b, ln2_s, ln2_b, w_h, b_h, alpha, beta, fc1_W, fc1_b,
           sources, destinations, neighbors, timestamps):
    idx2d = jnp.concatenate(
        [sources[:, None], destinations[:, None], neighbors], axis=1
    ).astype(jnp.int32).reshape(B * R // IDXC, IDXC)
    g = _sc_gather_combine(table, idx2d)

    t_f = timestamps.astype(jnp.float32).reshape(B, 1)
    bf = jnp.bfloat16

    def largs(l):
        return (Wqkv[l].astype(bf), bqkv[l].reshape(1, 3 * D),
                Wo[l].astype(bf), bo[l].reshape(1, D),
                Wf1[l].astype(bf), bf1[l].reshape(1, D),
                Wf2[l].astype(bf), bf2[l].reshape(1, D),
                ln1_s[l].reshape(1, D), ln1_b[l].reshape(1, D),
                ln2_s[l].reshape(1, D), ln2_b[l].reshape(1, D))

    h = _embed(g, W_emb.astype(bf), b_emb.reshape(1, D), t_f)
    h = _layer0(h, *largs(0))
    cls, ell, nll = _layer1_head(h, *largs(1), t_f, w_h, b_h.reshape(1, 1),
                                 alpha.reshape(1, 1), beta.reshape(1, 1),
                                 fc1_W, fc1_b.reshape(1, 2))
    return cls, ell.reshape(()), nll.reshape(())


# speedup vs baseline: 1.0873x; 1.0035x over previous
import functools
import math

import jax
import jax.numpy as jnp
from jax import lax
from jax.experimental import pallas as pl
from jax.experimental.pallas import tpu as pltpu
from jax.experimental.pallas import tpu_sc as plsc

N, D, B, K, NL, NH = 20000, 768, 1024, 16, 2, 2
R = K + 2
DH = D // NH
L = 16
NC, NS = 2, 16
NW = NC * NS
BPW = B // NW
E = 2
CH = BPW // E
IDXC = E * R
NBUF = 4


_sc_mesh = plsc.VectorSubcoreMesh(core_axis_name="c", subcore_axis_name="s")


@functools.partial(
    pl.kernel,
    mesh=_sc_mesh,
    out_type=jax.ShapeDtypeStruct((B, D), jnp.float32),
    scratch_types=[
        pltpu.VMEM((CH, IDXC), jnp.int32),
        pltpu.VMEM((IDXC, D), jnp.float32),
        pltpu.VMEM((IDXC, D), jnp.float32),
        pltpu.VMEM((IDXC, D), jnp.float32),
        pltpu.VMEM((IDXC, D), jnp.float32),
        pltpu.VMEM((E, D), jnp.float32),
        pltpu.VMEM((E, D), jnp.float32),
        pltpu.SemaphoreType.DMA,
        pltpu.SemaphoreType.DMA,
        pltpu.SemaphoreType.DMA,
        pltpu.SemaphoreType.DMA,
        pltpu.SemaphoreType.DMA,
        pltpu.SemaphoreType.DMA,
    ],
)
def _sc_gather_combine(table_hbm, idx_hbm, out_hbm, idx_v, rows0, rows1,
                       rows2, rows3, acc0, acc1, semg0, semg1, semg2, semg3,
                       semo0, semo1):
    wid = lax.axis_index("s") * NC + lax.axis_index("c")
    rows = (rows0, rows1, rows2, rows3)
    accs = (acc0, acc1)
    semg = (semg0, semg1, semg2, semg3)
    semo = (semo0, semo1)

    pltpu.sync_copy(idx_hbm.at[pl.ds(wid * CH, CH)], idx_v)
    for b in range(NBUF - 1):
        pltpu.async_copy(table_hbm.at[idx_v.at[b]], rows[b], semg[b])

    def quad(g, _):
        for b in range(NBUF):
            ci = NBUF * g + b
            a = b % 2

            @pl.when(ci + NBUF - 1 < CH)
            def _():
                pltpu.async_copy(
                    table_hbm.at[idx_v.at[ci + NBUF - 1]],
                    rows[(b + NBUF - 1) % NBUF], semg[(b + NBUF - 1) % NBUF])

            pltpu.make_async_copy(
                table_hbm.at[idx_v.at[ci]], rows[b], semg[b]).wait()

            @pl.when(ci >= 2)
            def _():
                pltpu.make_async_copy(
                    accs[a], out_hbm.at[pl.ds(wid * BPW, E)], semo[a]).wait()

            def lanes(c, _):
                off = c * L
                for e in range(E):
                    b0 = e * R
                    s = (rows[b][b0, pl.ds(off, L)]
                         + rows[b][b0 + 1, pl.ds(off, L)])
                    n = rows[b][b0 + 2, pl.ds(off, L)]
                    for j in range(3, R):
                        n = n + rows[b][b0 + j, pl.ds(off, L)]
                    accs[a][e, pl.ds(off, L)] = s + n * (1.0 / K)
                return 0

            lax.fori_loop(0, D // L, lanes, 0)
            pltpu.async_copy(
                accs[a], out_hbm.at[pl.ds((wid * BPW + ci * E), E)], semo[a])
        return 0

    lax.fori_loop(0, CH // NBUF, quad, 0)
    for a in range(2):
        pltpu.make_async_copy(
            accs[a], out_hbm.at[pl.ds(wid * BPW, E)], semo[a]).wait()


def _layernorm(x, s, b):
    m = x.mean(-1, keepdims=True)
    v = x.var(-1, keepdims=True)
    return (x - m) / jnp.sqrt(v + 1e-5) * s + b


def _embed_body(g_ref, w_ref, b_ref, t_ref, o_ref):
    x = jnp.tanh(
        jnp.dot(g_ref[...].astype(jnp.bfloat16), w_ref[...],
                preferred_element_type=jnp.float32)
        + b_ref[...]
    )
    i = lax.broadcasted_iota(jnp.int32, (B, D), 1)
    expo = (2.0 * (i // 2).astype(jnp.float32)) / D
    inv = jnp.exp(-math.log(10000.0) * expo)
    ang = (t_ref[...] / 100.0) * inv
    enc = jnp.where(i % 2 == 0, jnp.sin(ang), jnp.cos(ang))
    o_ref[...] = (x + enc).astype(jnp.bfloat16)


_embed = pl.pallas_call(
    _embed_body,
    out_shape=jax.ShapeDtypeStruct((B, D), jnp.bfloat16),
)


def _encoder_layer(h, wqkv_ref, bqkv_ref, wo_ref, bo_ref, wf1_ref, bf1_ref,
                   wf2_ref, bf2_ref, l1s_ref, l1b_ref, l2s_ref, l2b_ref):
    x = _layernorm(h, l1s_ref[...], l1b_ref[...]).astype(jnp.bfloat16)
    qkv = (jnp.dot(x, wqkv_ref[...], preferred_element_type=jnp.float32)
           + bqkv_ref[...])
    ri = lax.broadcasted_iota(jnp.int32, (B, B), 0)
    ci = lax.broadcasted_iota(jnp.int32, (B, B), 1)
    causal = ri >= ci
    parts = []
    for hh in range(NH):
        q = (qkv[:, hh * DH:(hh + 1) * DH]
             * (1.0 / math.sqrt(float(DH)))).astype(jnp.bfloat16)
        k = qkv[:, D + hh * DH:D + (hh + 1) * DH].astype(jnp.bfloat16)
        v = qkv[:, 2 * D + hh * DH:2 * D + (hh + 1) * DH].astype(jnp.bfloat16)
        sc = lax.dot_general(q, k, (((1,), (1,)), ((), ())),
                             preferred_element_type=jnp.float32)
        p = jnp.where(causal, jnp.exp(sc), 0.0)
        s = p.sum(-1, keepdims=True)
        o = jnp.dot(p.astype(jnp.bfloat16), v,
                    preferred_element_type=jnp.float32)
        parts.append(o * (1.0 / s))
    o = jnp.concatenate(parts, axis=1).astype(jnp.bfloat16)
    h = h + jnp.dot(o, wo_ref[...], preferred_element_type=jnp.float32) + bo_ref[...]
    y = _layernorm(h, l2s_ref[...], l2b_ref[...]).astype(jnp.bfloat16)
    f = jnp.maximum(
        jnp.dot(y, wf1_ref[...], preferred_element_type=jnp.float32) + bf1_ref[...],
        0.0).astype(jnp.bfloat16)
    h = h + jnp.dot(f, wf2_ref[...], preferred_element_type=jnp.float32) + bf2_ref[...]
    return h


def _layer0_body(h_ref, wqkv_ref, bqkv_ref, wo_ref, bo_ref, wf1_ref, bf1_ref,
                 wf2_ref, bf2_ref, l1s_ref, l1b_ref, l2s_ref, l2b_ref, o_ref):
    h = _encoder_layer(h_ref[...].astype(jnp.float32), wqkv_ref, bqkv_ref,
                       wo_ref, bo_ref, wf1_ref, bf1_ref, wf2_ref, bf2_ref,
                       l1s_ref, l1b_ref, l2s_ref, l2b_ref)
    o_ref[...] = h.astype(jnp.bfloat16)


_layer0 = pl.pallas_call(
    _layer0_body,
    out_shape=jax.ShapeDtypeStruct((B, D), jnp.bfloat16),
)


def _softplus(x):
    return jnp.maximum(x, 0.0) + jnp.log1p(jnp.exp(-jnp.abs(x)))


def _layer1_head_body(h_ref, wqkv_ref, bqkv_ref, wo_ref, bo_ref, wf1_ref,
                      bf1_ref, wf2_ref, bf2_ref, l1s_ref, l1b_ref, l2s_ref,
                      l2b_ref, t_ref, wh_ref, bh_ref, alpha_ref, beta_ref,
                      fw_ref, fb_ref, cls_ref, ell_ref, nll_ref):
    h = _encoder_layer(h_ref[...].astype(jnp.float32), wqkv_ref, bqkv_ref,
                       wo_ref, bo_ref, wf1_ref, bf1_ref, wf2_ref, bf2_ref,
                       l1s_ref, l1b_ref, l2s_ref, l2b_ref)
    t = t_ref[...]
    lam_in = (jnp.dot(h, wh_ref[...], preferred_element_type=jnp.float32)
              + bh_ref[...])
    tprev = jnp.concatenate([t[0:1], t[0:B - 1]], axis=0)
    dt = t - tprev
    idx = lax.broadcasted_iota(jnp.int32, (B, 1), 0)
    valid = idx >= 2
    lam = _softplus(lam_in + alpha_ref[...] * dt)
    event_ll = jnp.sum(jnp.where(valid, jnp.log(lam + 1e-9), 0.0))
    sp_beta = _softplus(beta_ref[...])
    non_event_ll = jnp.sum(jnp.where(valid, sp_beta * lam * dt, 0.0))
    hl = jnp.tanh(h[B - 1:B, :])
    logits = jnp.dot(hl, fw_ref[...], preferred_element_type=jnp.float32) + fb_ref[...]
    logits = logits - logits.max(-1, keepdims=True)
    p = jnp.exp(logits)
    cls_ref[...] = p / p.sum(-1, keepdims=True)
    ell_ref[...] = jnp.reshape(event_ll, (1, 1))
    nll_ref[...] = jnp.reshape(non_event_ll, (1, 1))


_layer1_head = pl.pallas_call(
    _layer1_head_body,
    out_shape=[
        jax.ShapeDtypeStruct((1, 2), jnp.float32),
        jax.ShapeDtypeStruct((1, 1), jnp.float32),
        jax.ShapeDtypeStruct((1, 1), jnp.float32),
    ],
)


def kernel(table, W_emb, b_emb, Wqkv, bqkv, Wo, bo, Wf1, bf1, Wf2, bf2,
           ln1_s, ln1_b, ln2_s, ln2_b, w_h, b_h, alpha, beta, fc1_W, fc1_b,
           sources, destinations, neighbors, timestamps):
    idx2d = jnp.concatenate(
        [sources[:, None], destinations[:, None], neighbors], axis=1
    ).astype(jnp.int32).reshape(B * R // IDXC, IDXC)
    g = _sc_gather_combine(table, idx2d)

    t_f = timestamps.astype(jnp.float32).reshape(B, 1)
    bf = jnp.bfloat16

    def largs(l):
        return (Wqkv[l].astype(bf), bqkv[l].reshape(1, 3 * D),
                Wo[l].astype(bf), bo[l].reshape(1, D),
                Wf1[l].astype(bf), bf1[l].reshape(1, D),
                Wf2[l].astype(bf), bf2[l].reshape(1, D),
                ln1_s[l].reshape(1, D), ln1_b[l].reshape(1, D),
                ln2_s[l].reshape(1, D), ln2_b[l].reshape(1, D))

    h = _embed(g, W_emb.astype(bf), b_emb.reshape(1, D), t_f)
    h = _layer0(h, *largs(0))
    cls, ell, nll = _layer1_head(h, *largs(1), t_f, w_h, b_h.reshape(1, 1),
                                 alpha.reshape(1, 1), beta.reshape(1, 1),
                                 fc1_W, fc1_b.reshape(1, 2))
    return cls, ell.reshape(()), nll.reshape(())
